# baseline (speedup 1.0000x reference)
import jax
import jax.numpy as jnp
from jax import lax
from jax.experimental import pallas as pl
from jax.experimental.pallas import tpu as pltpu
from jax.experimental.pallas import tpu_sc as plsc

_N = 10000
_DEG = 32
_D = 128
_MED_IDX = (_DEG - 1) // 2


def _oddeven_merge(lo, n, r):
    m = r * 2
    if m < n:
        yield from _oddeven_merge(lo, n, m)
        yield from _oddeven_merge(lo + r, n, m)
        for i in range(lo + r, lo + n - r, m):
            yield (i, i + r)
    else:
        yield (lo, lo + r)


def _oddeven_merge_sort(lo, hi):
    if (hi - lo) >= 1:
        mid = lo + ((hi - lo) // 2)
        yield from _oddeven_merge_sort(lo, mid)
        yield from _oddeven_merge_sort(mid + 1, hi)
        yield from _oddeven_merge(lo, hi - lo + 1, 1)


def _median_network(n, target):
    comps = list(_oddeven_merge_sort(0, n - 1))
    needed = {target}
    kept = []
    for (i, j) in reversed(comps):
        ni, nj = i in needed, j in needed
        if not ni and not nj:
            continue
        kept.append((i, j, 2 if (ni and nj) else (0 if ni else 1)))
        needed.add(i)
        needed.add(j)
    kept.reverse()
    return kept


_MED_OPS = _median_network(_DEG, _MED_IDX)

_MM_BLOCK = 2048
_MED_BLOCK = 400
_GW = 40


def _matmul_body(x_ref, w_ref, o_ref):
    o_ref[...] = jax.lax.dot_general(
        x_ref[...].astype(jnp.bfloat16), w_ref[...].astype(jnp.bfloat16),
        (((1,), (1,)), ((), ())),
        preferred_element_type=jnp.float32)


def _median_body(g_ref, o_ref):
    def chunk(c, carry):
        sl = pl.ds(c * 16, 16)
        v = [g_ref[k, sl, :].astype(jnp.bfloat16) for k in range(_DEG)]
        for (i, j, m) in _MED_OPS:
            a, b = v[i], v[j]
            if m == 2:
                v[i] = jnp.minimum(a, b)
                v[j] = jnp.maximum(a, b)
            elif m == 0:
                v[i] = jnp.minimum(a, b)
            else:
                v[j] = jnp.maximum(a, b)
        o_ref[sl, :] = v[_MED_IDX].astype(jnp.float32)
        return carry

    jax.lax.fori_loop(0, _MED_BLOCK // 16, chunk, 0)


def _sc_gather(h, idx2d):
    nwork, per = idx2d.shape
    d = h.shape[1]
    nwin = per // _GW
    sstage = h.shape[0] // 16
    mesh = plsc.VectorSubcoreMesh(core_axis_name="c", subcore_axis_name="s")

    @pl.kernel(
        out_type=jax.ShapeDtypeStruct((nwork * per, d), h.dtype),
        mesh=mesh,
        scratch_types=[
            pltpu.VMEM_SHARED((h.shape[0], d), h.dtype),
            pltpu.VMEM((per,), jnp.int32),
            pltpu.VMEM((2, _GW, d), h.dtype),
            pltpu.SemaphoreType.DMA,
            pltpu.SemaphoreType.DMA,
            pltpu.SemaphoreType.DMA,
        ],
    )
    def gather_kernel(h_hbm, i_hbm, o_hbm, h_spm, idx_v, rows_v,
                      sem_st, gsem_a, gsem_b):
        cc = lax.axis_index("c")
        ss = lax.axis_index("s")
        wid = cc * 16 + ss
        base = wid * per

        st = pl.ds(ss * sstage, sstage)
        pltpu.async_copy(h_hbm.at[st], h_spm.at[st], sem_st).wait()
        plsc.subcore_barrier()
        pltpu.async_copy(i_hbm.at[wid], idx_v, sem_st).wait()

        def g_src(w):
            return h_spm.at[idx_v.at[pl.ds(w * _GW, _GW)]]

        pltpu.async_copy(g_src(0), rows_v.at[0], gsem_a)

        @pl.loop(0, nwin - (nwin % 2), step=2)
        def _(w):
            pltpu.make_async_copy(g_src(w), rows_v.at[0], gsem_a).wait()

            pltpu.async_copy(g_src(w + 1), rows_v.at[1], gsem_b)
            pltpu.sync_copy(rows_v.at[0], o_hbm.at[pl.ds(base + w * _GW, _GW)])

            pltpu.make_async_copy(g_src(w + 1), rows_v.at[1], gsem_b).wait()

            @pl.when(w + 2 < nwin)
            def _():
                pltpu.async_copy(g_src(w + 2), rows_v.at[0], gsem_a)

            pltpu.sync_copy(rows_v.at[1],
                            o_hbm.at[pl.ds(base + (w + 1) * _GW, _GW)])

        if nwin % 2:
            w_last = nwin - 1
            pltpu.make_async_copy(g_src(w_last), rows_v.at[0], gsem_a).wait()
            pltpu.sync_copy(rows_v.at[0],
                            o_hbm.at[pl.ds(base + w_last * _GW, _GW)])

    return gather_kernel(h, idx2d)


_C = 2


def kernel(x, nbrs, W):
    n, d_in = x.shape
    deg = nbrs.shape[1]
    d_out = W.shape[0]
    per = n // _C

    npad = ((n + 2047) // 2048) * 2048

    h = pl.pallas_call(
        _matmul_body,
        grid=(npad // _MM_BLOCK,),
        in_specs=[
            pl.BlockSpec((_MM_BLOCK, d_in), lambda i: (i, 0)),
            pl.BlockSpec((d_out, d_in), lambda i: (0, 0)),
        ],
        out_specs=pl.BlockSpec((_MM_BLOCK, d_out), lambda i: (i, 0)),
        out_shape=jax.ShapeDtypeStruct((npad, d_out), jnp.float32),
    )(x, W)

    idx_t = nbrs.T
    outs = []
    for c in range(_C):
        idx_c = idx_t[:, c * per:(c + 1) * per]
        g = _sc_gather(h, idx_c)
        g3 = g.reshape(deg, per, d_out)
        outs.append(pl.pallas_call(
            _median_body,
            grid=(per // _MED_BLOCK,),
            in_specs=[pl.BlockSpec((deg, _MED_BLOCK, d_out),
                                   lambda i: (0, i, 0))],
            out_specs=pl.BlockSpec((_MED_BLOCK, d_out), lambda i: (i, 0)),
            out_shape=jax.ShapeDtypeStruct((per, d_out), jnp.float32),
        )(g3))
    return jnp.concatenate(outs, axis=0)


# speedup vs baseline: 1.0704x; 1.0704x over previous
import jax
import jax.numpy as jnp
from jax import lax
from jax.experimental import pallas as pl
from jax.experimental.pallas import tpu as pltpu
from jax.experimental.pallas import tpu_sc as plsc

_N = 10000
_DEG = 32
_D = 128
_MED_IDX = (_DEG - 1) // 2


def _oddeven_merge(lo, n, r):
    m = r * 2
    if m < n:
        yield from _oddeven_merge(lo, n, m)
        yield from _oddeven_merge(lo + r, n, m)
        for i in range(lo + r, lo + n - r, m):
            yield (i, i + r)
    else:
        yield (lo, lo + r)


def _oddeven_merge_sort(lo, hi):
    if (hi - lo) >= 1:
        mid = lo + ((hi - lo) // 2)
        yield from _oddeven_merge_sort(lo, mid)
        yield from _oddeven_merge_sort(mid + 1, hi)
        yield from _oddeven_merge(lo, hi - lo + 1, 1)


def _median_network(n, target):
    comps = list(_oddeven_merge_sort(0, n - 1))
    needed = {target}
    kept = []
    for (i, j) in reversed(comps):
        ni, nj = i in needed, j in needed
        if not ni and not nj:
            continue
        kept.append((i, j, 2 if (ni and nj) else (0 if ni else 1)))
        needed.add(i)
        needed.add(j)
    kept.reverse()
    return kept


_MED_OPS = _median_network(_DEG, _MED_IDX)
_SORT16 = list(_oddeven_merge_sort(0, _DEG // 2 - 1))

_MM_BLOCK = 2048
_MED_BLOCK = 400
_GW = 80


def _matmul_body(x_ref, w_ref, o_ref):
    o_ref[...] = jax.lax.dot_general(
        x_ref[...].astype(jnp.bfloat16), w_ref[...].astype(jnp.bfloat16),
        (((1,), (1,)), ((), ())),
        preferred_element_type=jnp.float32)


def _apply_net(v):
    h2 = _DEG // 2
    for (i, j) in _SORT16:
        a, b = v[i], v[j]
        v[i] = jnp.minimum(a, b)
        v[j] = jnp.maximum(a, b)
    for (i, j) in _SORT16:
        a, b = v[h2 + i], v[h2 + j]
        v[h2 + i] = jnp.minimum(a, b)
        v[h2 + j] = jnp.maximum(a, b)
    t = [jnp.minimum(v[k], v[_DEG - 1 - k]) for k in range(h2)]
    while len(t) > 1:
        t = [jnp.maximum(t[2 * i], t[2 * i + 1]) for i in range(len(t) // 2)]
    return t[0]


def _median_body(g_ref, o_ref):
    def chunk(c, carry):
        sl = pl.ds(c * 16, 16)
        v = [g_ref[k, sl, :].astype(jnp.bfloat16) for k in range(_DEG)]
        o_ref[sl, :] = _apply_net(v).astype(jnp.float32)
        return carry

    jax.lax.fori_loop(0, _MED_BLOCK // 16, chunk, 0)


def _sc_gather(h, idx2d):
    nwork, per = idx2d.shape
    d = h.shape[1]
    nwin = per // _GW
    sstage = h.shape[0] // 16
    mesh = plsc.VectorSubcoreMesh(core_axis_name="c", subcore_axis_name="s")

    @pl.kernel(
        out_type=jax.ShapeDtypeStruct((nwork * per, d), h.dtype),
        mesh=mesh,
        scratch_types=[
            pltpu.VMEM_SHARED((h.shape[0], d), h.dtype),
            pltpu.VMEM((per,), jnp.int32),
            pltpu.VMEM((2, _GW, d), h.dtype),
            pltpu.SemaphoreType.DMA,
            pltpu.SemaphoreType.DMA,
            pltpu.SemaphoreType.DMA,
        ],
    )
    def gather_kernel(h_hbm, i_hbm, o_hbm, h_spm, idx_v, rows_v,
                      sem_st, gsem_a, gsem_b):
        cc = lax.axis_index("c")
        ss = lax.axis_index("s")
        wid = cc * 16 + ss
        base = wid * per

        st = pl.ds(ss * sstage, sstage)
        pltpu.async_copy(h_hbm.at[st], h_spm.at[st], sem_st).wait()
        plsc.subcore_barrier()
        pltpu.async_copy(i_hbm.at[wid], idx_v, sem_st).wait()

        def g_src(w):
            return h_spm.at[idx_v.at[pl.ds(w * _GW, _GW)]]

        pltpu.async_copy(g_src(0), rows_v.at[0], gsem_a)

        @pl.loop(0, nwin - (nwin % 2), step=2)
        def _(w):
            pltpu.make_async_copy(g_src(w), rows_v.at[0], gsem_a).wait()

            pltpu.async_copy(g_src(w + 1), rows_v.at[1], gsem_b)
            pltpu.sync_copy(rows_v.at[0], o_hbm.at[pl.ds(base + w * _GW, _GW)])

            pltpu.make_async_copy(g_src(w + 1), rows_v.at[1], gsem_b).wait()

            @pl.when(w + 2 < nwin)
            def _():
                pltpu.async_copy(g_src(w + 2), rows_v.at[0], gsem_a)

            pltpu.sync_copy(rows_v.at[1],
                            o_hbm.at[pl.ds(base + (w + 1) * _GW, _GW)])

        if nwin % 2:
            w_last = nwin - 1
            pltpu.make_async_copy(g_src(w_last), rows_v.at[0], gsem_a).wait()
            pltpu.sync_copy(rows_v.at[0],
                            o_hbm.at[pl.ds(base + w_last * _GW, _GW)])

    return gather_kernel(h, idx2d)


_C = 1


def kernel(x, nbrs, W):
    n, d_in = x.shape
    deg = nbrs.shape[1]
    d_out = W.shape[0]
    per = n // _C

    npad = ((n + 2047) // 2048) * 2048

    h = pl.pallas_call(
        _matmul_body,
        grid=(npad // _MM_BLOCK,),
        in_specs=[
            pl.BlockSpec((_MM_BLOCK, d_in), lambda i: (i, 0)),
            pl.BlockSpec((d_out, d_in), lambda i: (0, 0)),
        ],
        out_specs=pl.BlockSpec((_MM_BLOCK, d_out), lambda i: (i, 0)),
        out_shape=jax.ShapeDtypeStruct((npad, d_out), jnp.float32),
        compiler_params=pltpu.CompilerParams(
            dimension_semantics=("parallel",)),
    )(x, W)

    idx_t = nbrs.T
    outs = []
    for c in range(_C):
        idx_c = idx_t[:, c * per:(c + 1) * per]
        g = _sc_gather(h, idx_c)
        g3 = g.reshape(deg, per, d_out)
        outs.append(pl.pallas_call(
            _median_body,
            grid=(per // _MED_BLOCK,),
            in_specs=[pl.BlockSpec((deg, _MED_BLOCK, d_out),
                                   lambda i: (0, i, 0))],
            out_specs=pl.BlockSpec((_MED_BLOCK, d_out), lambda i: (i, 0)),
            out_shape=jax.ShapeDtypeStruct((per, d_out), jnp.float32),
            compiler_params=pltpu.CompilerParams(
                dimension_semantics=("parallel",)),
        )(g3))
    return jnp.concatenate(outs, axis=0)


# speedup vs baseline: 1.0717x; 1.0013x over previous
import jax
import jax.numpy as jnp
from jax import lax
from jax.experimental import pallas as pl
from jax.experimental.pallas import tpu as pltpu
from jax.experimental.pallas import tpu_sc as plsc

_DEG = 32


def _oddeven_merge(lo, n, r):
    m = r * 2
    if m < n:
        yield from _oddeven_merge(lo, n, m)
        yield from _oddeven_merge(lo + r, n, m)
        for i in range(lo + r, lo + n - r, m):
            yield (i, i + r)
    else:
        yield (lo, lo + r)


def _oddeven_merge_sort(lo, hi):
    if (hi - lo) >= 1:
        mid = lo + ((hi - lo) // 2)
        yield from _oddeven_merge_sort(lo, mid)
        yield from _oddeven_merge_sort(mid + 1, hi)
        yield from _oddeven_merge(lo, hi - lo + 1, 1)


_SORT16 = list(_oddeven_merge_sort(0, _DEG // 2 - 1))

_MM_BLOCK = 2048
_MED_BLOCK = 400
_GW = 80


def _matmul_body(x_ref, w_ref, o_ref):
    o_ref[...] = jax.lax.dot_general(
        x_ref[...].astype(jnp.bfloat16), w_ref[...].astype(jnp.bfloat16),
        (((1,), (1,)), ((), ())),
        preferred_element_type=jnp.float32)


def _apply_net(v):
    h2 = _DEG // 2
    for (i, j) in _SORT16:
        a, b = v[i], v[j]
        v[i] = jnp.minimum(a, b)
        v[j] = jnp.maximum(a, b)
    for (i, j) in _SORT16:
        a, b = v[h2 + i], v[h2 + j]
        v[h2 + i] = jnp.minimum(a, b)
        v[h2 + j] = jnp.maximum(a, b)
    t = [jnp.minimum(v[k], v[_DEG - 1 - k]) for k in range(h2)]
    while len(t) > 1:
        t = [jnp.maximum(t[2 * i], t[2 * i + 1]) for i in range(len(t) // 2)]
    return t[0]


def _median_body(g_ref, o_ref):
    def chunk(c, carry):
        sl = pl.ds(c * 16, 16)
        v = [g_ref[k, sl, :].astype(jnp.bfloat16) for k in range(_DEG)]
        o_ref[sl, :] = _apply_net(v).astype(jnp.float32)
        return carry

    jax.lax.fori_loop(0, _MED_BLOCK // 16, chunk, 0)


def _sc_gather(h, idx2d):
    nwork, per = idx2d.shape
    d = h.shape[1]
    nwin = per // _GW
    sstage = h.shape[0] // 16
    mesh = plsc.VectorSubcoreMesh(core_axis_name="c", subcore_axis_name="s")

    @pl.kernel(
        out_type=jax.ShapeDtypeStruct((nwork * per, d), h.dtype),
        mesh=mesh,
        scratch_types=[
            pltpu.VMEM_SHARED((h.shape[0], d), h.dtype),
            pltpu.VMEM((per,), jnp.int32),
            pltpu.VMEM((2, _GW, d), h.dtype),
            pltpu.SemaphoreType.DMA,
            pltpu.SemaphoreType.DMA,
            pltpu.SemaphoreType.DMA,
        ],
    )
    def gather_kernel(h_hbm, i_hbm, o_hbm, h_spm, idx_v, rows_v,
                      sem_st, gsem_a, gsem_b):
        cc = lax.axis_index("c")
        ss = lax.axis_index("s")
        wid = cc * 16 + ss
        base = wid * per

        st = pl.ds(ss * sstage, sstage)
        pltpu.async_copy(h_hbm.at[st], h_spm.at[st], sem_st).wait()
        plsc.subcore_barrier()
        pltpu.async_copy(i_hbm.at[wid], idx_v, sem_st).wait()

        def g_src(w):
            return h_spm.at[idx_v.at[pl.ds(w * _GW, _GW)]]

        pltpu.async_copy(g_src(0), rows_v.at[0], gsem_a)

        @pl.loop(0, nwin - (nwin % 2), step=2)
        def _(w):
            pltpu.make_async_copy(g_src(w), rows_v.at[0], gsem_a).wait()

            pltpu.async_copy(g_src(w + 1), rows_v.at[1], gsem_b)
            pltpu.sync_copy(rows_v.at[0], o_hbm.at[pl.ds(base + w * _GW, _GW)])

            pltpu.make_async_copy(g_src(w + 1), rows_v.at[1], gsem_b).wait()

            @pl.when(w + 2 < nwin)
            def _():
                pltpu.async_copy(g_src(w + 2), rows_v.at[0], gsem_a)

            pltpu.sync_copy(rows_v.at[1],
                            o_hbm.at[pl.ds(base + (w + 1) * _GW, _GW)])

        if nwin % 2:
            w_last = nwin - 1
            pltpu.make_async_copy(g_src(w_last), rows_v.at[0], gsem_a).wait()
            pltpu.sync_copy(rows_v.at[0],
                            o_hbm.at[pl.ds(base + w_last * _GW, _GW)])

    return gather_kernel(h, idx2d)


_C = 1


def kernel(x, nbrs, W):
    n, d_in = x.shape
    deg = nbrs.shape[1]
    d_out = W.shape[0]
    per = n // _C

    npad = ((n + 2047) // 2048) * 2048

    h = pl.pallas_call(
        _matmul_body,
        grid=(npad // _MM_BLOCK,),
        in_specs=[
            pl.BlockSpec((_MM_BLOCK, d_in), lambda i: (i, 0)),
            pl.BlockSpec((d_out, d_in), lambda i: (0, 0)),
        ],
        out_specs=pl.BlockSpec((_MM_BLOCK, d_out), lambda i: (i, 0)),
        out_shape=jax.ShapeDtypeStruct((npad, d_out), jnp.float32),
        compiler_params=pltpu.CompilerParams(
            dimension_semantics=("parallel",)),
    )(x, W)

    idx_t = nbrs.T
    outs = []
    for c in range(_C):
        idx_c = idx_t[:, c * per:(c + 1) * per]
        g = _sc_gather(h, idx_c)
        g3 = g.reshape(deg, per, d_out)
        outs.append(pl.pallas_call(
            _median_body,
            grid=(per // _MED_BLOCK,),
            in_specs=[pl.BlockSpec((deg, _MED_BLOCK, d_out),
                                   lambda i: (0, i, 0))],
            out_specs=pl.BlockSpec((_MED_BLOCK, d_out), lambda i: (i, 0)),
            out_shape=jax.ShapeDtypeStruct((per, d_out), jnp.float32),
            compiler_params=pltpu.CompilerParams(
                dimension_semantics=("parallel",)),
        )(g3))
    return jnp.concatenate(outs, axis=0)
